# baseline (speedup 1.0000x reference)
import jax
import jax.numpy as jnp
from jax.experimental import pallas as pl
from jax.experimental.pallas import tpu as pltpu

DIM = 4096
N_EXPERTS = 64
TOPK = 8
BT = 1024


def _gate_kernel(x_ref, w_ref, wout_ref, iout_ref):
    x = x_ref[...]
    w = w_ref[...]
    scores = jax.lax.dot_general(
        w, x, (((1,), (1,)), ((), ())), preferred_element_type=jnp.float32
    )
    m = jnp.max(scores, axis=0, keepdims=True)
    e = jnp.exp(scores - m)
    inv_denom = 1.0 / jnp.sum(e, axis=0, keepdims=True)

    iota = jax.lax.broadcasted_iota(jnp.int32, e.shape, 0)
    s = e
    vals, idxs = [], []
    for k in range(TOPK):
        mx = jnp.max(s, axis=0, keepdims=True)
        idx = jnp.min(jnp.where(s == mx, iota, N_EXPERTS), axis=0, keepdims=True)
        vals.append(mx)
        idxs.append(idx)
        if k + 1 < TOPK:
            s = jnp.where(iota == idx, -1.0, s)
    wout_ref[...] = jnp.concatenate(vals, axis=0) * inv_denom
    iout_ref[...] = jnp.concatenate(idxs, axis=0)


def kernel(x, weight):
    n_tokens = x.shape[0]
    grid = (n_tokens // BT,)
    wout_t, iout_t = pl.pallas_call(
        _gate_kernel,
        grid=grid,
        in_specs=[
            pl.BlockSpec((BT, DIM), lambda i: (i, 0)),
            pl.BlockSpec((N_EXPERTS, DIM), lambda i: (0, 0)),
        ],
        out_specs=[
            pl.BlockSpec((TOPK, BT), lambda i: (0, i)),
            pl.BlockSpec((TOPK, BT), lambda i: (0, i)),
        ],
        out_shape=[
            jax.ShapeDtypeStruct((TOPK, n_tokens), jnp.float32),
            jax.ShapeDtypeStruct((TOPK, n_tokens), jnp.int32),
        ],
    )(x, weight)
    return wout_t.T, iout_t.T
